# baseline (speedup 1.0000x reference)
import jax
import jax.numpy as jnp
from jax import lax
from jax.experimental import pallas as pl
from jax.experimental.pallas import tpu as pltpu
from jax.experimental.pallas import tpu_sc as plsc

B = 16384
N_DENSE = 13
NF = 26
VOCAB = 100000
ED = 16
NCOL = N_DENSE + NF

NC, NS, L = 2, 16, 16
NW = NC * NS
RPW = B // NW
BLK = 64
NBLK = RPW // BLK
IPB = NF * BLK
SEG = 128
NSEG = IPB // SEG


def _sc_body(inp_hbm, tab_hbm, w_hbm, bias_hbm, out_hbm,
             inp_v, idx_v, gbuf, accb, outb, wv, bv, sem):
  wid = lax.axis_index("s") * NC + lax.axis_index("c")
  base = wid * RPW
  pltpu.sync_copy(inp_hbm.at[pl.ds(base, RPW)], inp_v)
  pltpu.sync_copy(w_hbm, wv)
  pltpu.sync_copy(bias_hbm, bv)

  iota = lax.iota(jnp.int32, L)
  offs_a = iota * VOCAB
  offs_b = (iota + 10) * VOCAB

  def build_row(j, carry):
    a = inp_v[j, pl.ds(N_DENSE, L)].astype(jnp.int32) + offs_a
    b = inp_v[j, pl.ds(NCOL - L, L)].astype(jnp.int32) + offs_b
    idx_v[pl.ds(j * NF, L)] = a
    idx_v[pl.ds(j * NF + (NF - L), L)] = b
    return carry

  lax.fori_loop(0, RPW, build_row, 0)

  wvec = wv[...]
  bvec = bv[...]

  def do_block(b, carry):
    ib = b * IPB

    def issue(m, c):
      pltpu.async_copy(tab_hbm.at[idx_v.at[pl.ds(ib + m * SEG, SEG)]],
                       gbuf.at[pl.ds(m * SEG, SEG)], sem)
      return c

    lax.fori_loop(0, NSEG, issue, 0)
    pltpu.make_async_copy(tab_hbm.at[pl.ds(0, IPB)], gbuf, sem).wait()

    def row(j, c):
      acc = inp_v[b * BLK + j, pl.ds(0, L)] * wvec
      for f in range(NF):
        acc = acc + gbuf[j * NF + f, :]
      accb[j, :] = acc
      return c

    lax.fori_loop(0, BLK, row, 0)

    def grp(g, c):
      out_vec = bvec
      for l in range(L):
        out_vec = out_vec + plsc.load_gather(
            accb, [g * L + iota, iota * 0 + l])
      outb[pl.ds(b * BLK + g * L, L)] = out_vec
      return c

    lax.fori_loop(0, BLK // L, grp, 0)
    return carry

  lax.fori_loop(0, NBLK, do_block, 0)
  pltpu.sync_copy(outb, out_hbm.at[pl.ds(base, RPW)])


def kernel(inputs, emb_tables, dense_weight, bias):
  table = emb_tables.reshape(NF * VOCAB, ED)
  w_pad = jnp.concatenate(
      [dense_weight[:, 0], jnp.zeros((L - N_DENSE,), jnp.float32)])
  bias_vec = jnp.broadcast_to(bias, (L,))

  mesh = plsc.VectorSubcoreMesh(core_axis_name="c", subcore_axis_name="s")
  out = pl.kernel(
      _sc_body,
      out_type=jax.ShapeDtypeStruct((B,), jnp.float32),
      mesh=mesh,
      compiler_params=pltpu.CompilerParams(
          needs_layout_passes=False, use_tc_tiling_on_sc=False),
      scratch_types=[
          pltpu.VMEM((RPW, NCOL), jnp.float32),
          pltpu.VMEM((RPW * NF,), jnp.int32),
          pltpu.VMEM((IPB, ED), jnp.float32),
          pltpu.VMEM((BLK, ED), jnp.float32),
          pltpu.VMEM((RPW,), jnp.float32),
          pltpu.VMEM((L,), jnp.float32),
          pltpu.VMEM((L,), jnp.float32),
          pltpu.SemaphoreType.DMA,
      ],
  )(inputs, table, w_pad, bias_vec)
  return out[:, None]
